# Initial kernel scaffold; baseline (speedup 1.0000x reference)
import jax
import jax.numpy as jnp
from jax.experimental import pallas as pl


def kernel(x, nodes, edge_index, count2label, W_gg, W_ih, W_hh, b_ih, b_hh, Wi, bi, Wj, bj, Wmlp, bmlp, Wc, bc):
    raise NotImplementedError("write your pallas kernel here")


# speedup vs baseline: 8.9585x; 8.9585x over previous
import functools

import jax
import jax.numpy as jnp
from jax import lax
from jax.experimental import pallas as pl
from jax.experimental.pallas import tpu as pltpu
from jax.experimental.pallas import tpu_sc as plsc

B = 2
N = 10000
D = 128
C = 129
E = 320000
CP = 144
CA = 258
CAP = 288
HID = 256
NLAYERS = 3

NSUB = 16
EPS = E // NSUB
K = 80
NCH = EPS // K
SUP = 50
NSUP = NCH // SUP
RPS = 632
RPS_LAST = N - (NSUB - 1) * RPS

_f32 = jnp.float32


def _seg_body(h_hbm, srcg_hbm, dst_hbm, z_hbm, out_hbm,
              src_v, dst_v, buf0, buf1, acc, sem0, sem1):
    c = lax.axis_index("c")
    s = lax.axis_index("s")
    @pl.when(s < NSUB - 1)
    def _():
        pltpu.sync_copy(z_hbm.at[pl.ds(s * RPS, RPS)],
                        acc.at[pl.ds(s * RPS, RPS)])

    @pl.when(s == NSUB - 1)
    def _():
        pltpu.sync_copy(z_hbm.at[pl.ds((NSUB - 1) * RPS, RPS_LAST)],
                        acc.at[pl.ds((NSUB - 1) * RPS, RPS_LAST)])
    plsc.subcore_barrier()

    def superchunk(g, carry):
        pltpu.sync_copy(srcg_hbm.at[c, s, pl.ds(g * SUP, SUP)], src_v)
        pltpu.sync_copy(dst_hbm.at[s, pl.ds(g * SUP, SUP)], dst_v)
        pltpu.async_copy(h_hbm.at[src_v.at[0]], buf0, sem0)

        def body(j, carry2):
            i0 = j * 2
            pltpu.async_copy(h_hbm.at[src_v.at[i0 + 1]], buf1, sem1)
            pltpu.make_async_copy(h_hbm.at[src_v.at[i0]], buf0, sem0).wait()
            pltpu.sync_copy(buf0, acc.at[dst_v.at[i0]], add=True)

            @pl.when(i0 + 2 < SUP)
            def _():
                pltpu.async_copy(h_hbm.at[src_v.at[i0 + 2]], buf0, sem0)

            pltpu.make_async_copy(h_hbm.at[src_v.at[i0 + 1]], buf1, sem1).wait()
            pltpu.sync_copy(buf1, acc.at[dst_v.at[i0 + 1]], add=True)
            return carry2

        lax.fori_loop(0, SUP // 2, body, 0)
        return carry

    lax.fori_loop(0, NSUP, superchunk, 0)
    plsc.subcore_barrier()
    @pl.when(s < NSUB - 1)
    def _():
        pltpu.sync_copy(acc.at[pl.ds(s * RPS, RPS)],
                        out_hbm.at[pl.ds(c * N + s * RPS, RPS)])

    @pl.when(s == NSUB - 1)
    def _():
        pltpu.sync_copy(acc.at[pl.ds((NSUB - 1) * RPS, RPS_LAST)],
                        out_hbm.at[pl.ds(c * N + (NSUB - 1) * RPS, RPS_LAST)])


def _make_seg_kernel():
    mesh = plsc.VectorSubcoreMesh(core_axis_name="c", subcore_axis_name="s")
    return pl.kernel(
        _seg_body,
        mesh=mesh,
        compiler_params=pltpu.CompilerParams(use_tc_tiling_on_sc=False),
        out_type=jax.ShapeDtypeStruct((B * N, CP), _f32),
        scratch_types=[
            pltpu.VMEM((SUP, K), jnp.int32),
            pltpu.VMEM((SUP, K), jnp.int32),
            pltpu.VMEM((K, CP), _f32),
            pltpu.VMEM((K, CP), _f32),
            pltpu.VMEM_SHARED((N, CP), _f32),
            pltpu.SemaphoreType.DMA,
            pltpu.SemaphoreType.DMA,
        ],
    )


def _gru_body(S_ref, h_ref, Wgg_ref, Ur_ref, Uz_ref, Un_ref,
              Vr_ref, Vz_ref, Vn_ref, bias_ref, out_ref):
    dot = functools.partial(jnp.dot, preferred_element_type=_f32,
                            precision=jax.lax.Precision.HIGHEST)
    m = dot(S_ref[...], Wgg_ref[...])
    h = h_ref[...]
    b = bias_ref[...]
    r = jax.nn.sigmoid(dot(m, Ur_ref[...]) + dot(h, Vr_ref[...]) + b[0:1])
    z = jax.nn.sigmoid(dot(m, Uz_ref[...]) + dot(h, Vz_ref[...]) + b[1:2])
    n = jnp.tanh(dot(m, Un_ref[...]) + b[2:3] + r * (dot(h, Vn_ref[...]) + b[3:4]))
    out_ref[...] = (1.0 - z) * n + z * h


def _gru_call(S, h, Wgg, Ur, Uz, Un, Vr, Vz, Vn, bias, blk=2000):
    grid = (B * N) // blk
    row_spec = pl.BlockSpec((blk, CP), lambda i: (i, 0))
    w_spec = pl.BlockSpec((CP, CP), lambda i: (0, 0))
    return pl.pallas_call(
        _gru_body,
        grid=(grid,),
        in_specs=[row_spec, row_spec] + [w_spec] * 7
        + [pl.BlockSpec((4, CP), lambda i: (0, 0))],
        out_specs=row_spec,
        out_shape=jax.ShapeDtypeStruct((B * N, CP), _f32),
    )(S, h, Wgg, Ur, Uz, Un, Vr, Vz, Vn, bias)


def _att_body(h_ref, h0_ref, WiH_ref, WiN_ref, WjH_ref, WjN_ref, b_ref, out_ref):
    i = pl.program_id(0)

    @pl.when(i == 0)
    def _():
        out_ref[...] = jnp.zeros_like(out_ref)

    dot = functools.partial(jnp.dot, preferred_element_type=_f32,
                            precision=jax.lax.Precision.HIGHEST)
    h = h_ref[...]
    h0 = h0_ref[...]
    b = b_ref[...]
    t1 = jax.nn.sigmoid(dot(h, WiH_ref[...]) + dot(h0, WiN_ref[...]) + b[0:1])
    t2 = jax.nn.relu(dot(h, WjH_ref[...]) + dot(h0, WjN_ref[...]) + b[1:2])
    partial = jnp.sum(t1 * t2, axis=0, keepdims=True)
    graph = i // (N // h.shape[0])
    onehot = (lax.broadcasted_iota(jnp.int32, (B, 1), 0) == graph).astype(_f32)
    out_ref[...] += onehot * partial


def _att_call(h, h0, WiH, WiN, WjH, WjN, batt, blk=2000):
    grid = (B * N) // blk
    row_spec = pl.BlockSpec((blk, CP), lambda i: (i, 0))
    w_spec = pl.BlockSpec((CP, CAP), lambda i: (0, 0))
    return pl.pallas_call(
        _att_body,
        grid=(grid,),
        in_specs=[row_spec, row_spec] + [w_spec] * 4
        + [pl.BlockSpec((2, CAP), lambda i: (0, 0))],
        out_specs=pl.BlockSpec((B, CAP), lambda i: (0, 0)),
        out_shape=jax.ShapeDtypeStruct((B, CAP), _f32),
    )(h, h0, WiH, WiN, WjH, WjN, batt)


def _mlp_body(p_ref, Wm_ref, bm_ref, Wc_ref, bc_ref, out_ref):
    dot = functools.partial(jnp.dot, preferred_element_type=_f32,
                            precision=jax.lax.Precision.HIGHEST)
    p = jax.nn.relu(p_ref[...])
    state = jax.nn.relu(dot(p, Wm_ref[...]) + bm_ref[...])
    out_ref[...] = dot(state, Wc_ref[...]) + bc_ref[...]


def _mlp_call(pooled, Wm, bm, WcP, bcP):
    return pl.pallas_call(
        _mlp_body,
        out_shape=jax.ShapeDtypeStruct((B, 128), _f32),
    )(pooled, Wm, bm, WcP, bcP)


def _pad2(w, r, c):
    return jnp.zeros((r, c), w.dtype).at[: w.shape[0], : w.shape[1]].set(w)


def _padv(v, n, off=0):
    return jnp.zeros((n,), v.dtype).at[off: off + v.shape[0]].set(v)


def kernel(x, nodes, edge_index, count2label, W_gg, W_ih, W_hh, b_ih, b_hh,
           Wi, bi, Wj, bj, Wmlp, bmlp, Wc, bc):
    cov = jnp.zeros((B, N), x.dtype).at[:, count2label].set(x)
    h0p = jnp.zeros((B * N, CP), _f32)
    h0p = h0p.at[:, :D].set(jnp.tile(nodes, (B, 1)))
    h0p = h0p.at[:, D].set(cov.reshape(-1))

    src = edge_index[0]
    dst = edge_index[1]
    srcg = jnp.stack([src, src + N]).reshape(B, NSUB, NCH, K)
    dstr = dst.reshape(NSUB, NCH, K)
    zrows = jnp.zeros((N, CP), _f32)

    U = W_ih.T
    V = W_hh.T
    Ur, Uz, Un = [_pad2(U[:, i * C:(i + 1) * C], CP, CP) for i in range(3)]
    Vr, Vz, Vn = [_pad2(V[:, i * C:(i + 1) * C], CP, CP) for i in range(3)]
    bir, biz, bin_ = [_padv(b_ih[i * C:(i + 1) * C], CP) for i in range(3)]
    bhr, bhz, bhn = [_padv(b_hh[i * C:(i + 1) * C], CP) for i in range(3)]
    gbias = jnp.stack([bir + bhr, biz + bhz, bin_, bhn])
    WiT = Wi.T
    WjT = Wj.T
    WiH = _pad2(WiT[:C, :], CP, CAP)
    WiN = _pad2(WiT[C:, :], CP, CAP)
    WjH = _pad2(WjT[:C, :], CP, CAP)
    WjN = _pad2(WjT[C:, :], CP, CAP)
    batt = jnp.stack([_padv(bi, CAP), _padv(bj, CAP)])
    Wm = _pad2(Wmlp.T, CAP, HID)
    bm = bmlp.reshape(1, HID)
    WcP = _pad2(Wc.T, HID, 128)
    bcP = _pad2(bc.reshape(1, 1), 1, 128)

    seg = _make_seg_kernel()
    h = h0p
    for l in range(NLAYERS):
        S = seg(h, srcg, dstr, zrows)
        h = _gru_call(S, h, _pad2(W_gg[l], CP, CP), Ur, Uz, Un, Vr, Vz, Vn, gbias)
    pooled = _att_call(h, h0p, WiH, WiN, WjH, WjN, batt)
    out = _mlp_call(pooled, Wm, bm, WcP, bcP)
    return out[:, :1]


# speedup vs baseline: 9.3383x; 1.0424x over previous
import functools

import jax
import jax.numpy as jnp
import numpy as np
from jax import lax
from jax.experimental import pallas as pl
from jax.experimental.pallas import tpu as pltpu
from jax.experimental.pallas import tpu_sc as plsc

B = 2
N = 10000
D = 128
C = 129
E = 320000
CP = 144
CA = 258
CAP = 288
HID = 256
NLAYERS = 3

NSUB = 16
EPS = E // NSUB
K = 88
NPAD = 64
EPP = EPS + NPAD
NCH = EPP // K
SUP = 12
NSUP = NCH // SUP
NA = N + 8
RPS = 632
RPS_LAST = N - (NSUB - 1) * RPS

_f32 = jnp.float32


def _seg_body(h_hbm, srcg_hbm, dst_hbm, z_hbm, out_hbm,
              src_v, dst_v, buf0, buf1, buf2, acc,
              sg0, sg1, sg2, ss0, ss1, ss2):
    c = lax.axis_index("c")
    s = lax.axis_index("s")
    @pl.when(s < NSUB - 1)
    def _():
        pltpu.sync_copy(z_hbm.at[pl.ds(s * RPS, RPS)],
                        acc.at[pl.ds(s * RPS, RPS)])

    @pl.when(s == NSUB - 1)
    def _():
        pltpu.sync_copy(z_hbm.at[pl.ds((NSUB - 1) * RPS, RPS_LAST)],
                        acc.at[pl.ds((NSUB - 1) * RPS, RPS_LAST)])
    plsc.subcore_barrier()

    bufs = (buf0, buf1, buf2)
    sgs = (sg0, sg1, sg2)
    sss = (ss0, ss1, ss2)

    def g_start(ci, b):
        pltpu.async_copy(h_hbm.at[src_v.at[ci]], bufs[b], sgs[b])

    def g_wait(ci, b):
        pltpu.make_async_copy(h_hbm.at[src_v.at[ci]], bufs[b], sgs[b]).wait()

    def s_start(ci, b):
        pltpu.async_copy(bufs[b], acc.at[dst_v.at[ci]], sss[b], add=True)

    def s_wait(ci, b):
        pltpu.make_async_copy(bufs[b], acc.at[dst_v.at[ci]], sss[b]).wait()

    def superchunk(g, carry):
        pltpu.sync_copy(srcg_hbm.at[c, s, pl.ds(g * SUP, SUP)], src_v)
        pltpu.sync_copy(dst_hbm.at[s, pl.ds(g * SUP, SUP)], dst_v)
        g_start(0, 0)
        g_start(1, 1)

        def chunk(ci, b):
            g_wait(ci, b)
            s_start(ci, b)

            @pl.when(ci + 2 < SUP)
            def _():
                @pl.when(ci >= 1)
                def _():
                    s_wait(ci - 1, (b + 2) % 3)
                g_start(ci + 2, (b + 2) % 3)

        def body(j, carry2):
            c0 = j * 3
            chunk(c0, 0)
            chunk(c0 + 1, 1)
            chunk(c0 + 2, 2)
            return carry2

        lax.fori_loop(0, SUP // 3, body, 0)
        s_wait(SUP - 3, 0)
        s_wait(SUP - 2, 1)
        s_wait(SUP - 1, 2)
        return carry

    lax.fori_loop(0, NSUP, superchunk, 0)
    plsc.subcore_barrier()
    @pl.when(s < NSUB - 1)
    def _():
        pltpu.sync_copy(acc.at[pl.ds(s * RPS, RPS)],
                        out_hbm.at[pl.ds(c * N + s * RPS, RPS)])

    @pl.when(s == NSUB - 1)
    def _():
        pltpu.sync_copy(acc.at[pl.ds((NSUB - 1) * RPS, RPS_LAST)],
                        out_hbm.at[pl.ds(c * N + (NSUB - 1) * RPS, RPS_LAST)])


def _make_seg_kernel():
    mesh = plsc.VectorSubcoreMesh(core_axis_name="c", subcore_axis_name="s")
    return pl.kernel(
        _seg_body,
        mesh=mesh,
        compiler_params=pltpu.CompilerParams(use_tc_tiling_on_sc=False),
        out_type=jax.ShapeDtypeStruct((B * N, CP), _f32),
        scratch_types=[
            pltpu.VMEM((SUP, K), jnp.int32),
            pltpu.VMEM((SUP, K), jnp.int32),
            pltpu.VMEM((K, CP), _f32),
            pltpu.VMEM((K, CP), _f32),
            pltpu.VMEM((K, CP), _f32),
            pltpu.VMEM_SHARED((NA, CP), _f32),
            pltpu.SemaphoreType.DMA,
            pltpu.SemaphoreType.DMA,
            pltpu.SemaphoreType.DMA,
            pltpu.SemaphoreType.DMA,
            pltpu.SemaphoreType.DMA,
            pltpu.SemaphoreType.DMA,
        ],
    )


def _gru_body(S_ref, h_ref, Ur_ref, Uz_ref, Un_ref,
              Vr_ref, Vz_ref, Vn_ref, bias_ref, out_ref):
    dot = functools.partial(jnp.dot, preferred_element_type=_f32,
                            precision=jax.lax.Precision.HIGHEST)
    s = S_ref[...]
    h = h_ref[...]
    b = bias_ref[...]
    r = jax.nn.sigmoid(dot(s, Ur_ref[...]) + dot(h, Vr_ref[...]) + b[0:1])
    z = jax.nn.sigmoid(dot(s, Uz_ref[...]) + dot(h, Vz_ref[...]) + b[1:2])
    n = jnp.tanh(dot(s, Un_ref[...]) + b[2:3] + r * (dot(h, Vn_ref[...]) + b[3:4]))
    out_ref[...] = (1.0 - z) * n + z * h


def _gru_call(S, h, Ur, Uz, Un, Vr, Vz, Vn, bias, blk=2000):
    grid = (B * N) // blk
    row_spec = pl.BlockSpec((blk, CP), lambda i: (i, 0))
    w_spec = pl.BlockSpec((CP, CP), lambda i: (0, 0))
    return pl.pallas_call(
        _gru_body,
        grid=(grid,),
        in_specs=[row_spec, row_spec] + [w_spec] * 6
        + [pl.BlockSpec((4, CP), lambda i: (0, 0))],
        out_specs=row_spec,
        out_shape=jax.ShapeDtypeStruct((B * N, CP), _f32),
    )(S, h, Ur, Uz, Un, Vr, Vz, Vn, bias)


def _att_body(h_ref, h0_ref, WiH_ref, WiN_ref, WjH_ref, WjN_ref, b_ref, out_ref):
    i = pl.program_id(0)

    @pl.when(i == 0)
    def _():
        out_ref[...] = jnp.zeros_like(out_ref)

    dot = functools.partial(jnp.dot, preferred_element_type=_f32,
                            precision=jax.lax.Precision.HIGHEST)
    h = h_ref[...]
    h0 = h0_ref[...]
    b = b_ref[...]
    t1 = jax.nn.sigmoid(dot(h, WiH_ref[...]) + dot(h0, WiN_ref[...]) + b[0:1])
    t2 = jax.nn.relu(dot(h, WjH_ref[...]) + dot(h0, WjN_ref[...]) + b[1:2])
    partial = jnp.sum(t1 * t2, axis=0, keepdims=True)
    graph = i // (N // h.shape[0])
    onehot = (lax.broadcasted_iota(jnp.int32, (B, 1), 0) == graph).astype(_f32)
    out_ref[...] += onehot * partial


def _att_call(h, h0, WiH, WiN, WjH, WjN, batt, blk=2000):
    grid = (B * N) // blk
    row_spec = pl.BlockSpec((blk, CP), lambda i: (i, 0))
    w_spec = pl.BlockSpec((CP, CAP), lambda i: (0, 0))
    return pl.pallas_call(
        _att_body,
        grid=(grid,),
        in_specs=[row_spec, row_spec] + [w_spec] * 4
        + [pl.BlockSpec((2, CAP), lambda i: (0, 0))],
        out_specs=pl.BlockSpec((B, CAP), lambda i: (0, 0)),
        out_shape=jax.ShapeDtypeStruct((B, CAP), _f32),
    )(h, h0, WiH, WiN, WjH, WjN, batt)


def _mlp_body(p_ref, Wm_ref, bm_ref, Wc_ref, bc_ref, out_ref):
    dot = functools.partial(jnp.dot, preferred_element_type=_f32,
                            precision=jax.lax.Precision.HIGHEST)
    p = jax.nn.relu(p_ref[...])
    state = jax.nn.relu(dot(p, Wm_ref[...]) + bm_ref[...])
    out_ref[...] = dot(state, Wc_ref[...]) + bc_ref[...]


def _mlp_call(pooled, Wm, bm, WcP, bcP):
    return pl.pallas_call(
        _mlp_body,
        out_shape=jax.ShapeDtypeStruct((B, 128), _f32),
    )(pooled, Wm, bm, WcP, bcP)


def _pad2(w, r, c):
    return jnp.zeros((r, c), w.dtype).at[: w.shape[0], : w.shape[1]].set(w)


def _padv(v, n, off=0):
    return jnp.zeros((n,), v.dtype).at[off: off + v.shape[0]].set(v)


def kernel(x, nodes, edge_index, count2label, W_gg, W_ih, W_hh, b_ih, b_hh,
           Wi, bi, Wj, bj, Wmlp, bmlp, Wc, bc):
    cov = jnp.zeros((B, N), x.dtype).at[:, count2label].set(x)
    h0p = jnp.zeros((B * N, CP), _f32)
    h0p = h0p.at[:, :D].set(jnp.tile(nodes, (B, 1)))
    h0p = h0p.at[:, D].set(cov.reshape(-1))

    src = edge_index[0].reshape(NSUB, EPS)
    dst = edge_index[1].reshape(NSUB, EPS)
    sn = np.arange(NSUB)[:, None]
    jn = np.arange(NPAD)[None, :]
    pad_src = jnp.asarray((1237 * sn + 631 * jn) % (B * N), dtype=jnp.int32)
    pad_dst = jnp.asarray(N + (sn + jn) % 8, dtype=jnp.int32)
    srcg = jnp.stack([
        jnp.concatenate([src + c * N, pad_src], axis=1) for c in range(B)
    ]).reshape(B, NSUB, NCH, K)
    dstr = jnp.concatenate([dst, pad_dst], axis=1).reshape(NSUB, NCH, K)
    zrows = jnp.zeros((N, CP), _f32)

    U = W_ih.T
    V = W_hh.T
    Ur, Uz, Un = [_pad2(U[:, i * C:(i + 1) * C], CP, CP) for i in range(3)]
    Vr, Vz, Vn = [_pad2(V[:, i * C:(i + 1) * C], CP, CP) for i in range(3)]
    bir, biz, bin_ = [_padv(b_ih[i * C:(i + 1) * C], CP) for i in range(3)]
    bhr, bhz, bhn = [_padv(b_hh[i * C:(i + 1) * C], CP) for i in range(3)]
    gbias = jnp.stack([bir + bhr, biz + bhz, bin_, bhn])
    WiT = Wi.T
    WjT = Wj.T
    WiH = _pad2(WiT[:C, :], CP, CAP)
    WiN = _pad2(WiT[C:, :], CP, CAP)
    WjH = _pad2(WjT[:C, :], CP, CAP)
    WjN = _pad2(WjT[C:, :], CP, CAP)
    batt = jnp.stack([_padv(bi, CAP), _padv(bj, CAP)])
    Wm = _pad2(Wmlp.T, CAP, HID)
    bm = bmlp.reshape(1, HID)
    WcP = _pad2(Wc.T, HID, 128)
    bcP = _pad2(bc.reshape(1, 1), 1, 128)

    hp = functools.partial(jnp.matmul, preferred_element_type=_f32,
                           precision=jax.lax.Precision.HIGHEST)
    seg = _make_seg_kernel()
    h = h0p
    for l in range(NLAYERS):
        Wp = _pad2(W_gg[l], CP, CP)
        S = seg(h, srcg, dstr, zrows)
        h = _gru_call(S, h, hp(Wp, Ur), hp(Wp, Uz), hp(Wp, Un),
                      Vr, Vz, Vn, gbias)
    pooled = _att_call(h, h0p, WiH, WiN, WjH, WjN, batt)
    out = _mlp_call(pooled, Wm, bm, WcP, bcP)
    return out[:, :1]


# speedup vs baseline: 9.5076x; 1.0181x over previous
import functools

import jax
import jax.numpy as jnp
import numpy as np
from jax import lax
from jax.experimental import pallas as pl
from jax.experimental.pallas import tpu as pltpu
from jax.experimental.pallas import tpu_sc as plsc

B = 2
N = 10000
D = 128
C = 129
E = 320000
CP = 144
CA = 258
CAP = 288
HID = 256
NLAYERS = 3

NSUB = 16
EPS = E // NSUB
K = 88
NPAD = 64
EPP = EPS + NPAD
NCH = EPP // K
SUP = 12
NSUP = NCH // SUP
NA = N + 8
RPS = 632
RPS_LAST = N - (NSUB - 1) * RPS

_f32 = jnp.float32


def _seg_body(h_hbm, srcg_hbm, dst_hbm, z_hbm, out_hbm,
              src_v, dst_v, buf0, buf1, buf2, acc,
              sg0, sg1, sg2, ss0, ss1, ss2):
    c = lax.axis_index("c")
    s = lax.axis_index("s")
    @pl.when(s < NSUB - 1)
    def _():
        pltpu.sync_copy(z_hbm.at[pl.ds(s * RPS, RPS)],
                        acc.at[pl.ds(s * RPS, RPS)])

    @pl.when(s == NSUB - 1)
    def _():
        pltpu.sync_copy(z_hbm.at[pl.ds((NSUB - 1) * RPS, RPS_LAST)],
                        acc.at[pl.ds((NSUB - 1) * RPS, RPS_LAST)])
    plsc.subcore_barrier()

    bufs = (buf0, buf1, buf2)
    sgs = (sg0, sg1, sg2)
    sss = (ss0, ss1, ss2)

    def g_start(ci, b):
        pltpu.async_copy(h_hbm.at[src_v.at[ci]], bufs[b], sgs[b])

    def g_wait(ci, b):
        pltpu.make_async_copy(h_hbm.at[src_v.at[ci]], bufs[b], sgs[b]).wait()

    def s_start(ci, b):
        pltpu.async_copy(bufs[b], acc.at[dst_v.at[ci]], sss[b], add=True)

    def s_wait(ci, b):
        pltpu.make_async_copy(bufs[b], acc.at[dst_v.at[ci]], sss[b]).wait()

    def superchunk(g, carry):
        pltpu.sync_copy(srcg_hbm.at[c, s, pl.ds(g * SUP, SUP)], src_v)
        pltpu.sync_copy(dst_hbm.at[s, pl.ds(g * SUP, SUP)], dst_v)
        g_start(0, 0)
        g_start(1, 1)

        def chunk(ci, b):
            g_wait(ci, b)
            s_start(ci, b)

            @pl.when(ci + 2 < SUP)
            def _():
                @pl.when(ci >= 1)
                def _():
                    s_wait(ci - 1, (b + 2) % 3)
                g_start(ci + 2, (b + 2) % 3)

        def body(j, carry2):
            c0 = j * 3
            chunk(c0, 0)
            chunk(c0 + 1, 1)
            chunk(c0 + 2, 2)
            return carry2

        lax.fori_loop(0, SUP // 3, body, 0)
        s_wait(SUP - 3, 0)
        s_wait(SUP - 2, 1)
        s_wait(SUP - 1, 2)
        return carry

    lax.fori_loop(0, NSUP, superchunk, 0)
    plsc.subcore_barrier()
    @pl.when(s < NSUB - 1)
    def _():
        pltpu.sync_copy(acc.at[pl.ds(s * RPS, RPS)],
                        out_hbm.at[pl.ds(c * N + s * RPS, RPS)])

    @pl.when(s == NSUB - 1)
    def _():
        pltpu.sync_copy(acc.at[pl.ds((NSUB - 1) * RPS, RPS_LAST)],
                        out_hbm.at[pl.ds(c * N + (NSUB - 1) * RPS, RPS_LAST)])


def _make_seg_kernel():
    mesh = plsc.VectorSubcoreMesh(core_axis_name="c", subcore_axis_name="s")
    return pl.kernel(
        _seg_body,
        mesh=mesh,
        compiler_params=pltpu.CompilerParams(use_tc_tiling_on_sc=False),
        out_type=jax.ShapeDtypeStruct((B * N, CP), _f32),
        scratch_types=[
            pltpu.VMEM((SUP, K), jnp.int32),
            pltpu.VMEM((SUP, K), jnp.int32),
            pltpu.VMEM((K, CP), _f32),
            pltpu.VMEM((K, CP), _f32),
            pltpu.VMEM((K, CP), _f32),
            pltpu.VMEM_SHARED((NA, CP), _f32),
            pltpu.SemaphoreType.DMA,
            pltpu.SemaphoreType.DMA,
            pltpu.SemaphoreType.DMA,
            pltpu.SemaphoreType.DMA,
            pltpu.SemaphoreType.DMA,
            pltpu.SemaphoreType.DMA,
        ],
    )


def _gru_body(S_ref, h_ref, Ur_ref, Uz_ref, Un_ref,
              Vr_ref, Vz_ref, Vn_ref, bias_ref, out_ref):
    dot = functools.partial(jnp.dot, preferred_element_type=_f32,
                            precision=jax.lax.Precision.HIGHEST)
    s = S_ref[...]
    h = h_ref[...]
    b = bias_ref[...]
    r = jax.nn.sigmoid(dot(s, Ur_ref[...]) + dot(h, Vr_ref[...]) + b[0:1])
    z = jax.nn.sigmoid(dot(s, Uz_ref[...]) + dot(h, Vz_ref[...]) + b[1:2])
    n = jnp.tanh(dot(s, Un_ref[...]) + b[2:3] + r * (dot(h, Vn_ref[...]) + b[3:4]))
    out_ref[...] = (1.0 - z) * n + z * h


def _gru_call(S, h, Ur, Uz, Un, Vr, Vz, Vn, bias, blk=2000):
    grid = (B * N) // blk
    row_spec = pl.BlockSpec((blk, CP), lambda i: (i, 0))
    w_spec = pl.BlockSpec((CP, CP), lambda i: (0, 0))
    return pl.pallas_call(
        _gru_body,
        grid=(grid,),
        in_specs=[row_spec, row_spec] + [w_spec] * 6
        + [pl.BlockSpec((4, CP), lambda i: (0, 0))],
        out_specs=row_spec,
        out_shape=jax.ShapeDtypeStruct((B * N, CP), _f32),
    )(S, h, Ur, Uz, Un, Vr, Vz, Vn, bias)


def _att_body(h_ref, h0_ref, WiH_ref, WiN_ref, WjH_ref, WjN_ref, b_ref, out_ref):
    i = pl.program_id(0)

    @pl.when(i == 0)
    def _():
        out_ref[...] = jnp.zeros_like(out_ref)

    dot = functools.partial(jnp.dot, preferred_element_type=_f32,
                            precision=jax.lax.Precision.HIGHEST)
    h = h_ref[...]
    h0 = h0_ref[...]
    b = b_ref[...]
    t1 = jax.nn.sigmoid(dot(h, WiH_ref[...]) + dot(h0, WiN_ref[...]) + b[0:1])
    t2 = jax.nn.relu(dot(h, WjH_ref[...]) + dot(h0, WjN_ref[...]) + b[1:2])
    partial = jnp.sum(t1 * t2, axis=0, keepdims=True)
    graph = i // (N // h.shape[0])
    onehot = (lax.broadcasted_iota(jnp.int32, (B, 1), 0) == graph).astype(_f32)
    out_ref[...] += onehot * partial


def _att_call(h, h0, WiH, WiN, WjH, WjN, batt, blk=2000):
    grid = (B * N) // blk
    row_spec = pl.BlockSpec((blk, CP), lambda i: (i, 0))
    w_spec = pl.BlockSpec((CP, CAP), lambda i: (0, 0))
    return pl.pallas_call(
        _att_body,
        grid=(grid,),
        in_specs=[row_spec, row_spec] + [w_spec] * 4
        + [pl.BlockSpec((2, CAP), lambda i: (0, 0))],
        out_specs=pl.BlockSpec((B, CAP), lambda i: (0, 0)),
        out_shape=jax.ShapeDtypeStruct((B, CAP), _f32),
    )(h, h0, WiH, WiN, WjH, WjN, batt)


def _asm_body(nodes_ref, cov_ref, out_ref):
    blk = nodes_ref.shape[0]
    out_ref[...] = jnp.concatenate(
        [nodes_ref[...], cov_ref[...], jnp.zeros((blk, CP - D - 1), _f32)],
        axis=1)


def _asm_call(nodes, cov_col, blk=2000):
    grid = (B * N) // blk
    per_graph = N // blk
    return pl.pallas_call(
        _asm_body,
        grid=(grid,),
        in_specs=[
            pl.BlockSpec((blk, D), lambda i, pg=per_graph: (i % pg, 0)),
            pl.BlockSpec((blk, 1), lambda i: (i, 0)),
        ],
        out_specs=pl.BlockSpec((blk, CP), lambda i: (i, 0)),
        out_shape=jax.ShapeDtypeStruct((B * N, CP), _f32),
    )(nodes, cov_col)


def _mlp_body(p_ref, Wm_ref, bm_ref, Wc_ref, bc_ref, out_ref):
    dot = functools.partial(jnp.dot, preferred_element_type=_f32,
                            precision=jax.lax.Precision.HIGHEST)
    p = jax.nn.relu(p_ref[...])
    state = jax.nn.relu(dot(p, Wm_ref[...]) + bm_ref[...])
    out_ref[...] = dot(state, Wc_ref[...]) + bc_ref[...]


def _mlp_call(pooled, Wm, bm, WcP, bcP):
    return pl.pallas_call(
        _mlp_body,
        out_shape=jax.ShapeDtypeStruct((B, 128), _f32),
    )(pooled, Wm, bm, WcP, bcP)


def _pad2(w, r, c):
    return jnp.zeros((r, c), w.dtype).at[: w.shape[0], : w.shape[1]].set(w)


def _padv(v, n, off=0):
    return jnp.zeros((n,), v.dtype).at[off: off + v.shape[0]].set(v)


def kernel(x, nodes, edge_index, count2label, W_gg, W_ih, W_hh, b_ih, b_hh,
           Wi, bi, Wj, bj, Wmlp, bmlp, Wc, bc):
    cov = jnp.zeros((B, N), x.dtype).at[:, count2label].set(x)
    h0p = _asm_call(nodes, cov.reshape(-1, 1))

    src = edge_index[0].reshape(NSUB, EPS)
    dst = edge_index[1].reshape(NSUB, EPS)
    sn = np.arange(NSUB)[:, None]
    jn = np.arange(NPAD)[None, :]
    pad_src = jnp.asarray((1237 * sn + 631 * jn) % (B * N), dtype=jnp.int32)
    pad_dst = jnp.asarray(N + (sn + jn) % 8, dtype=jnp.int32)
    srcg = jnp.stack([
        jnp.concatenate([src + c * N, pad_src], axis=1) for c in range(B)
    ]).reshape(B, NSUB, NCH, K)
    dstr = jnp.concatenate([dst, pad_dst], axis=1).reshape(NSUB, NCH, K)
    zrows = jnp.zeros((N, CP), _f32)

    U = W_ih.T
    V = W_hh.T
    Ur, Uz, Un = [_pad2(U[:, i * C:(i + 1) * C], CP, CP) for i in range(3)]
    Vr, Vz, Vn = [_pad2(V[:, i * C:(i + 1) * C], CP, CP) for i in range(3)]
    bir, biz, bin_ = [_padv(b_ih[i * C:(i + 1) * C], CP) for i in range(3)]
    bhr, bhz, bhn = [_padv(b_hh[i * C:(i + 1) * C], CP) for i in range(3)]
    gbias = jnp.stack([bir + bhr, biz + bhz, bin_, bhn])
    WiT = Wi.T
    WjT = Wj.T
    WiH = _pad2(WiT[:C, :], CP, CAP)
    WiN = _pad2(WiT[C:, :], CP, CAP)
    WjH = _pad2(WjT[:C, :], CP, CAP)
    WjN = _pad2(WjT[C:, :], CP, CAP)
    batt = jnp.stack([_padv(bi, CAP), _padv(bj, CAP)])
    Wm = _pad2(Wmlp.T, CAP, HID)
    bm = bmlp.reshape(1, HID)
    WcP = _pad2(Wc.T, HID, 128)
    bcP = _pad2(bc.reshape(1, 1), 1, 128)

    hp = functools.partial(jnp.matmul, preferred_element_type=_f32,
                           precision=jax.lax.Precision.HIGHEST)
    seg = _make_seg_kernel()
    h = h0p
    for l in range(NLAYERS):
        Wp = _pad2(W_gg[l], CP, CP)
        S = seg(h, srcg, dstr, zrows)
        h = _gru_call(S, h, hp(Wp, Ur), hp(Wp, Uz), hp(Wp, Un),
                      Vr, Vz, Vn, gbias)
    pooled = _att_call(h, h0p, WiH, WiN, WjH, WjN, batt)
    out = _mlp_call(pooled, Wm, bm, WcP, bcP)
    return out[:, :1]
